# baseline (speedup 1.0000x reference)
import functools

import jax
import jax.numpy as jnp
from jax import lax
from jax.experimental import pallas as pl
from jax.experimental.pallas import tpu as pltpu
from jax.experimental.pallas import tpu_sc as plsc

EMBED = 256
HEADS = 8
LEVELS = 4
POINTS = 4
DPH = EMBED // HEADS
SHAPES = [[92, 160], [46, 80], [23, 40], [12, 20]]
NV = sum(h * w for h, w in SHAPES)
BS = 4
NQ = 900
NW = 32
ROWS_PER_Q = LEVELS * POINTS * 4
Q_PER_GROUP = 2
GROUP_ROWS = ROWS_PER_Q * Q_PER_GROUP
GROUPS = NQ // Q_PER_GROUP
CHUNK = 24
GROUPS_PAD = 456
N_CHUNKS = GROUPS_PAD // CHUNK
NQ_PAD = GROUPS_PAD * Q_PER_GROUP


def _matmul_bias_kernel(x_ref, w_ref, b_ref, o_ref, *, cast_bf16):
    x = x_ref[...]
    w = w_ref[...]
    if cast_bf16:
        x = x.astype(jnp.bfloat16)
        w = w.astype(jnp.bfloat16)
    o_ref[...] = (
        jnp.dot(x, w, preferred_element_type=jnp.float32) + b_ref[...]
    ).astype(o_ref.dtype)


def _matmul_bias(x, w, b, block_m, out_dtype=jnp.float32, cast_bf16=False):
    m, k = x.shape
    n = w.shape[1]
    assert m % block_m == 0
    return pl.pallas_call(
        functools.partial(_matmul_bias_kernel, cast_bf16=cast_bf16),
        grid=(m // block_m,),
        in_specs=[
            pl.BlockSpec((block_m, k), lambda i: (i, 0)),
            pl.BlockSpec((k, n), lambda i: (0, 0)),
            pl.BlockSpec((1, n), lambda i: (0, 0)),
        ],
        out_specs=pl.BlockSpec((block_m, n), lambda i: (i, 0)),
        out_shape=jax.ShapeDtypeStruct((m, n), out_dtype),
    )(x, w, b.reshape(1, n))


def _sc_gather_weighted_sum(table, idx, wts):
    mesh = plsc.VectorSubcoreMesh(core_axis_name="c", subcore_axis_name="s")

    @functools.partial(
        pl.kernel,
        out_type=jax.ShapeDtypeStruct((NW, NQ, DPH), jnp.float32),
        mesh=mesh,
        scratch_types=[
            pltpu.VMEM((CHUNK, GROUP_ROWS), jnp.int32),
            pltpu.VMEM((CHUNK, GROUP_ROWS), jnp.float32),
            pltpu.VMEM((GROUP_ROWS, DPH), jnp.bfloat16),
            pltpu.VMEM((GROUP_ROWS, DPH), jnp.bfloat16),
            pltpu.VMEM((NQ_PAD, DPH), jnp.float32),
            pltpu.SemaphoreType.DMA,
            pltpu.SemaphoreType.DMA,
        ],
        compiler_params=pltpu.CompilerParams(use_tc_tiling_on_sc=False, needs_layout_passes=False),
    )
    def sc_kernel(table_hbm, idx_hbm, w_hbm, out_hbm,
                  idx_v, w_v, buf0, buf1, out_v, sem0, sem1):
        wid = lax.axis_index("s") * 2 + lax.axis_index("c")
        bufs = (buf0, buf1)
        sems = (sem0, sem1)

        def compute_group(g_local, g_abs, buf):
            for sub in range(Q_PER_GROUP):
                acc_e = jnp.zeros((16,), jnp.float32)
                acc_o = jnp.zeros((16,), jnp.float32)
                for jc in range(ROWS_PER_Q // 16):
                    base = sub * ROWS_PER_Q + jc * 16
                    w16 = w_v[g_local, pl.ds(base, 16)]
                    for k in range(16):
                        raw = plsc.bitcast(buf[base + k, :], jnp.int32)
                        even = plsc.bitcast(raw << 16, jnp.float32)
                        odd = plsc.bitcast(raw & jnp.int32(-65536), jnp.float32)
                        acc_e = acc_e + w16[k] * even
                        acc_o = acc_o + w16[k] * odd
                q_local = g_abs * Q_PER_GROUP + sub
                out_v[q_local, pl.ds(0, 16)] = acc_e
                out_v[q_local, pl.ds(16, 16)] = acc_o

        def chunk_body(c, _):
            pltpu.sync_copy(idx_hbm.at[wid, pl.ds(c * CHUNK, CHUNK)], idx_v)
            pltpu.sync_copy(w_hbm.at[wid, pl.ds(c * CHUNK, CHUNK)], w_v)
            pltpu.async_copy(table_hbm.at[idx_v.at[0]], bufs[0], sems[0])

            def pair_body(i2, _):
                g0 = 2 * i2
                g1 = g0 + 1
                pltpu.async_copy(table_hbm.at[idx_v.at[g1]], bufs[1], sems[1])
                pltpu.make_async_copy(
                    table_hbm.at[idx_v.at[g0]], bufs[0], sems[0]).wait()
                compute_group(g0, c * CHUNK + g0, bufs[0])
                @pl.when(i2 < CHUNK // 2 - 1)
                def _():
                    pltpu.async_copy(
                        table_hbm.at[idx_v.at[g0 + 2]], bufs[0], sems[0])
                pltpu.make_async_copy(
                    table_hbm.at[idx_v.at[g1]], bufs[1], sems[1]).wait()
                compute_group(g1, c * CHUNK + g1, bufs[1])
                return 0

            lax.fori_loop(0, CHUNK // 2, pair_body, 0)
            return 0

        lax.fori_loop(0, N_CHUNKS, chunk_body, 0)
        pltpu.sync_copy(out_v.at[pl.ds(0, NQ)], out_hbm.at[wid])

    return sc_kernel(table, idx, wts)


def _build_indices_weights(reference_points, off, aw):
    shapes = jnp.array(SHAPES, dtype=jnp.float32)
    wh = jnp.stack([shapes[:, 1], shapes[:, 0]], axis=-1)
    loc = reference_points[:, :, None, :, None, :] + off / wh[None, None, None, :, None, :]
    x = loc[..., 0] * wh[None, None, None, :, None, 0] - 0.5
    y = loc[..., 1] * wh[None, None, None, :, None, 1] - 0.5
    x0 = jnp.floor(x)
    y0 = jnp.floor(y)
    fx = x - x0
    fy = y - y0
    Wl = wh[None, None, None, :, None, 0]
    Hl = wh[None, None, None, :, None, 1]
    starts = []
    s = 0
    for (H_, W_) in SHAPES:
        starts.append(s)
        s += H_ * W_
    lvl_start = jnp.array(starts, dtype=jnp.float32)[None, None, None, :, None]
    lvl_w = wh[None, None, None, :, None, 0]

    idx_c = []
    wts_c = []
    for (dy, dx, wexpr) in (
            (0.0, 0.0, lambda: (1 - fx) * (1 - fy)),
            (1.0, 0.0, lambda: (1 - fx) * fy),
            (0.0, 1.0, lambda: fx * (1 - fy)),
            (1.0, 1.0, lambda: fx * fy)):
        ix = x0 + dx
        iy = y0 + dy
        valid = ((ix >= 0) & (ix <= Wl - 1) & (iy >= 0) & (iy <= Hl - 1))
        ixc = jnp.clip(ix, 0, Wl - 1)
        iyc = jnp.clip(iy, 0, Hl - 1)
        flat = lvl_start + iyc * lvl_w + ixc
        idx_c.append(flat)
        wts_c.append(wexpr() * valid.astype(jnp.float32))
    flat4 = jnp.stack(idx_c, axis=-1)
    w4 = jnp.stack(wts_c, axis=-1) * aw[..., None]
    b_ix = jnp.arange(BS, dtype=jnp.float32)[:, None, None, None, None, None]
    h_ix = jnp.arange(HEADS, dtype=jnp.float32)[None, None, :, None, None, None]
    rows = (b_ix * NV + flat4) * HEADS + h_ix
    rows = rows.astype(jnp.int32)
    rows = rows.reshape(BS, NQ, HEADS, ROWS_PER_Q).transpose(0, 2, 1, 3)
    w4 = w4.reshape(BS, NQ, HEADS, ROWS_PER_Q).transpose(0, 2, 1, 3)
    rows = rows.reshape(NW, GROUPS, GROUP_ROWS)
    w4 = w4.reshape(NW, GROUPS, GROUP_ROWS)
    pad = ((0, 0), (0, GROUPS_PAD - GROUPS), (0, 0))
    return jnp.pad(rows, pad), jnp.pad(w4, pad)


def kernel(query, value, reference_points, spatial_shapes, W_value, b_value,
           W_off, b_off, W_attn, b_attn, W_out, b_out):
    bs, nq, _ = query.shape
    nv = value.shape[1]

    v = _matmul_bias(value.reshape(bs * nv, EMBED), W_value, b_value,
                     block_m=480, out_dtype=jnp.bfloat16, cast_bf16=True)
    table = v.reshape(bs * nv * HEADS, DPH)

    q2 = query.reshape(bs * nq, EMBED)
    w_cat = jnp.concatenate([W_off, W_attn], axis=1)
    b_cat = jnp.concatenate([b_off, b_attn], axis=0)
    proj = _matmul_bias(q2, w_cat, b_cat, block_m=400)
    off = proj[:, : HEADS * LEVELS * POINTS * 2].reshape(
        bs, nq, HEADS, LEVELS, POINTS, 2)
    aw = proj[:, HEADS * LEVELS * POINTS * 2:].reshape(
        bs, nq, HEADS, LEVELS * POINTS)
    aw = jax.nn.softmax(aw, axis=-1).reshape(bs, nq, HEADS, LEVELS, POINTS)

    idx, wts = _build_indices_weights(reference_points, off, aw)
    sampled = _sc_gather_weighted_sum(table, idx, wts)
    sampled = sampled.reshape(bs, HEADS, nq, DPH).transpose(0, 2, 1, 3)

    perm32 = jnp.concatenate(
        [jnp.arange(0, DPH, 2), jnp.arange(1, DPH, 2)])
    W_out_p = W_out.reshape(HEADS, DPH, EMBED)[:, perm32, :].reshape(EMBED, EMBED)
    out = _matmul_bias(sampled.reshape(bs * nq, EMBED), W_out_p, b_out, block_m=400)
    return out.reshape(bs, nq, EMBED) + query


# speedup vs baseline: 1.0321x; 1.0321x over previous
import functools

import jax
import jax.numpy as jnp
from jax import lax
from jax.experimental import pallas as pl
from jax.experimental.pallas import tpu as pltpu
from jax.experimental.pallas import tpu_sc as plsc

EMBED = 256
HEADS = 8
LEVELS = 4
POINTS = 4
DPH = EMBED // HEADS
SHAPES = [[92, 160], [46, 80], [23, 40], [12, 20]]
NV = sum(h * w for h, w in SHAPES)
BS = 4
NQ = 900
NW = 32
ROWS_PER_Q = LEVELS * POINTS * 4
Q_PER_GROUP = 2
GROUP_ROWS = ROWS_PER_Q * Q_PER_GROUP
GROUPS = NQ // Q_PER_GROUP
CHUNK = 24
GROUPS_PAD = 456
N_CHUNKS = GROUPS_PAD // CHUNK
NQ_PAD = GROUPS_PAD * Q_PER_GROUP


def _matmul_bias_kernel(x_ref, w_ref, b_ref, o_ref, *, cast_bf16):
    x = x_ref[...]
    w = w_ref[...]
    if cast_bf16:
        x = x.astype(jnp.bfloat16)
        w = w.astype(jnp.bfloat16)
    o_ref[...] = (
        jnp.dot(x, w, preferred_element_type=jnp.float32) + b_ref[...]
    ).astype(o_ref.dtype)


def _matmul_bias(x, w, b, block_m, out_dtype=jnp.float32, cast_bf16=False):
    m, k = x.shape
    n = w.shape[1]
    assert m % block_m == 0
    return pl.pallas_call(
        functools.partial(_matmul_bias_kernel, cast_bf16=cast_bf16),
        grid=(m // block_m,),
        in_specs=[
            pl.BlockSpec((block_m, k), lambda i: (i, 0)),
            pl.BlockSpec((k, n), lambda i: (0, 0)),
            pl.BlockSpec((1, n), lambda i: (0, 0)),
        ],
        out_specs=pl.BlockSpec((block_m, n), lambda i: (i, 0)),
        out_shape=jax.ShapeDtypeStruct((m, n), out_dtype),
    )(x, w, b.reshape(1, n))


def _sc_gather_weighted_sum(table, idx, wts):
    mesh = plsc.VectorSubcoreMesh(core_axis_name="c", subcore_axis_name="s")

    @functools.partial(
        pl.kernel,
        out_type=jax.ShapeDtypeStruct((NW, NQ, DPH), jnp.float32),
        mesh=mesh,
        scratch_types=[
            pltpu.VMEM((CHUNK, GROUP_ROWS), jnp.int32),
            pltpu.VMEM((CHUNK, GROUP_ROWS), jnp.int32),
            pltpu.VMEM((CHUNK, GROUP_ROWS), jnp.float32),
            pltpu.VMEM((CHUNK, GROUP_ROWS), jnp.float32),
            pltpu.VMEM((GROUP_ROWS, DPH), jnp.bfloat16),
            pltpu.VMEM((GROUP_ROWS, DPH), jnp.bfloat16),
            pltpu.VMEM((NQ_PAD, DPH), jnp.float32),
            pltpu.SemaphoreType.DMA,
            pltpu.SemaphoreType.DMA,
            pltpu.SemaphoreType.DMA,
            pltpu.SemaphoreType.DMA,
        ],
        compiler_params=pltpu.CompilerParams(use_tc_tiling_on_sc=False, needs_layout_passes=False),
    )
    def sc_kernel(table_hbm, idx_hbm, w_hbm, out_hbm,
                  idx_va, idx_vb, w_va, w_vb, buf0, buf1, out_v,
                  sem0, sem1, ssem0, ssem1):
        wid = lax.axis_index("s") * 2 + lax.axis_index("c")
        bufs = (buf0, buf1)
        sems = (sem0, sem1)
        idx_stages = (idx_va, idx_vb)
        w_stages = (w_va, w_vb)
        ssems = (ssem0, ssem1)

        def stage_issue(c, par):
            pltpu.async_copy(
                idx_hbm.at[wid, pl.ds(c * CHUNK, CHUNK)], idx_stages[par],
                ssems[par])
            pltpu.async_copy(
                w_hbm.at[wid, pl.ds(c * CHUNK, CHUNK)], w_stages[par],
                ssems[par])

        def stage_wait(c, par):
            pltpu.make_async_copy(
                idx_hbm.at[wid, pl.ds(c * CHUNK, CHUNK)], idx_stages[par],
                ssems[par]).wait()
            pltpu.make_async_copy(
                w_hbm.at[wid, pl.ds(c * CHUNK, CHUNK)], w_stages[par],
                ssems[par]).wait()

        def compute_group(w_v, g_local, g_abs, buf):
            for sub in range(Q_PER_GROUP):
                acc_e = jnp.zeros((16,), jnp.float32)
                acc_o = jnp.zeros((16,), jnp.float32)
                for jc in range(ROWS_PER_Q // 16):
                    base = sub * ROWS_PER_Q + jc * 16
                    w16 = w_v[g_local, pl.ds(base, 16)]
                    for k in range(16):
                        raw = plsc.bitcast(buf[base + k, :], jnp.int32)
                        even = plsc.bitcast(raw << 16, jnp.float32)
                        odd = plsc.bitcast(raw & jnp.int32(-65536), jnp.float32)
                        acc_e = acc_e + w16[k] * even
                        acc_o = acc_o + w16[k] * odd
                q_local = g_abs * Q_PER_GROUP + sub
                out_v[q_local, pl.ds(0, 16)] = acc_e
                out_v[q_local, pl.ds(16, 16)] = acc_o

        def run_chunk(c, par):
            stage_wait(c, par)
            idx_v = idx_stages[par]
            w_v = w_stages[par]
            @pl.when(c + 1 < N_CHUNKS)
            def _():
                stage_issue(c + 1, 1 - par)
            pltpu.async_copy(table_hbm.at[idx_v.at[0]], bufs[0], sems[0])

            def pair_body(i2, _):
                g0 = 2 * i2
                g1 = g0 + 1
                pltpu.async_copy(table_hbm.at[idx_v.at[g1]], bufs[1], sems[1])
                pltpu.make_async_copy(
                    table_hbm.at[idx_v.at[g0]], bufs[0], sems[0]).wait()
                compute_group(w_v, g0, c * CHUNK + g0, bufs[0])
                @pl.when(i2 < CHUNK // 2 - 1)
                def _():
                    pltpu.async_copy(
                        table_hbm.at[idx_v.at[g0 + 2]], bufs[0], sems[0])
                pltpu.make_async_copy(
                    table_hbm.at[idx_v.at[g1]], bufs[1], sems[1]).wait()
                compute_group(w_v, g1, c * CHUNK + g1, bufs[1])
                return 0

            lax.fori_loop(0, CHUNK // 2, pair_body, 0)

        stage_issue(0, 0)

        def chunk_pair(c2, _):
            run_chunk(2 * c2, 0)
            @pl.when(2 * c2 + 1 < N_CHUNKS)
            def _():
                run_chunk(2 * c2 + 1, 1)
            return 0

        lax.fori_loop(0, (N_CHUNKS + 1) // 2, chunk_pair, 0)
        pltpu.sync_copy(out_v.at[pl.ds(0, NQ)], out_hbm.at[wid])

    return sc_kernel(table, idx, wts)


def _build_indices_weights(reference_points, off, aw):
    shapes = jnp.array(SHAPES, dtype=jnp.float32)
    wh = jnp.stack([shapes[:, 1], shapes[:, 0]], axis=-1)
    loc = reference_points[:, :, None, :, None, :] + off / wh[None, None, None, :, None, :]
    x = loc[..., 0] * wh[None, None, None, :, None, 0] - 0.5
    y = loc[..., 1] * wh[None, None, None, :, None, 1] - 0.5
    x0 = jnp.floor(x)
    y0 = jnp.floor(y)
    fx = x - x0
    fy = y - y0
    Wl = wh[None, None, None, :, None, 0]
    Hl = wh[None, None, None, :, None, 1]
    starts = []
    s = 0
    for (H_, W_) in SHAPES:
        starts.append(s)
        s += H_ * W_
    lvl_start = jnp.array(starts, dtype=jnp.float32)[None, None, None, :, None]
    lvl_w = wh[None, None, None, :, None, 0]

    idx_c = []
    wts_c = []
    for (dy, dx, wexpr) in (
            (0.0, 0.0, lambda: (1 - fx) * (1 - fy)),
            (1.0, 0.0, lambda: (1 - fx) * fy),
            (0.0, 1.0, lambda: fx * (1 - fy)),
            (1.0, 1.0, lambda: fx * fy)):
        ix = x0 + dx
        iy = y0 + dy
        valid = ((ix >= 0) & (ix <= Wl - 1) & (iy >= 0) & (iy <= Hl - 1))
        ixc = jnp.clip(ix, 0, Wl - 1)
        iyc = jnp.clip(iy, 0, Hl - 1)
        flat = lvl_start + iyc * lvl_w + ixc
        idx_c.append(flat)
        wts_c.append(wexpr() * valid.astype(jnp.float32))
    flat4 = jnp.stack(idx_c, axis=-1)
    w4 = jnp.stack(wts_c, axis=-1) * aw[..., None]
    b_ix = jnp.arange(BS, dtype=jnp.float32)[:, None, None, None, None, None]
    h_ix = jnp.arange(HEADS, dtype=jnp.float32)[None, None, :, None, None, None]
    rows = (b_ix * NV + flat4) * HEADS + h_ix
    rows = rows.astype(jnp.int32)
    rows = rows.reshape(BS, NQ, HEADS, ROWS_PER_Q).transpose(0, 2, 1, 3)
    w4 = w4.reshape(BS, NQ, HEADS, ROWS_PER_Q).transpose(0, 2, 1, 3)
    rows = rows.reshape(NW, GROUPS, GROUP_ROWS)
    w4 = w4.reshape(NW, GROUPS, GROUP_ROWS)
    pad = ((0, 0), (0, GROUPS_PAD - GROUPS), (0, 0))
    return jnp.pad(rows, pad), jnp.pad(w4, pad)


def kernel(query, value, reference_points, spatial_shapes, W_value, b_value,
           W_off, b_off, W_attn, b_attn, W_out, b_out):
    bs, nq, _ = query.shape
    nv = value.shape[1]

    v = _matmul_bias(value.reshape(bs * nv, EMBED), W_value, b_value,
                     block_m=480, out_dtype=jnp.bfloat16, cast_bf16=True)
    table = v.reshape(bs * nv * HEADS, DPH)

    q2 = query.reshape(bs * nq, EMBED)
    w_cat = jnp.concatenate([W_off, W_attn], axis=1)
    b_cat = jnp.concatenate([b_off, b_attn], axis=0)
    proj = _matmul_bias(q2, w_cat, b_cat, block_m=400)
    off = proj[:, : HEADS * LEVELS * POINTS * 2].reshape(
        bs, nq, HEADS, LEVELS, POINTS, 2)
    aw = proj[:, HEADS * LEVELS * POINTS * 2:].reshape(
        bs, nq, HEADS, LEVELS * POINTS)
    aw = jax.nn.softmax(aw, axis=-1).reshape(bs, nq, HEADS, LEVELS, POINTS)

    idx, wts = _build_indices_weights(reference_points, off, aw)
    sampled = _sc_gather_weighted_sum(table, idx, wts)
    sampled = sampled.reshape(bs, HEADS, nq, DPH).transpose(0, 2, 1, 3)

    perm32 = jnp.concatenate(
        [jnp.arange(0, DPH, 2), jnp.arange(1, DPH, 2)])
    W_out_p = W_out.reshape(HEADS, DPH, EMBED)[:, perm32, :].reshape(EMBED, EMBED)
    out = _matmul_bias(sampled.reshape(bs * nq, EMBED), W_out_p, b_out, block_m=400)
    return out.reshape(bs, nq, EMBED) + query
